# baseline (speedup 1.0000x reference)
import functools

import jax
import jax.numpy as jnp
from jax import lax
from jax.experimental import pallas as pl
from jax.experimental.pallas import tpu as pltpu
from jax.experimental.pallas import tpu_sc as plsc

LANES = 16
NC, NS = 2, 16
NW = NC * NS
CHUNK0 = 128
NBUF = 4
UNROLL = 8
VBLK = 4096


@functools.partial(jax.jit, static_argnames=("vocab", "d_model"))
def _tc_pack(table_t, vocab, d_model):
    per = 128 // d_model
    grid = vocab // VBLK

    def body(x_ref, o_ref):
        xt = x_ref[...].T
        v3 = xt.reshape(VBLK // 128, 128, d_model)
        a = v3[:, :64, :].reshape(VBLK // per, d_model)
        b = v3[:, 64:, :].reshape(VBLK // per, d_model)
        o_ref[...] = jnp.concatenate([a, b], axis=1)

    return pl.pallas_call(
        body,
        grid=(grid,),
        in_specs=[pl.BlockSpec((d_model, VBLK), lambda i: (0, i))],
        out_specs=pl.BlockSpec((VBLK // per, 128), lambda i: (i, 0)),
        out_shape=jax.ShapeDtypeStruct((vocab // per, 128), jnp.float32),
    )(table_t)


@functools.partial(jax.jit, static_argnames=())
def _tc_out_t(x):
    b, ld = x.shape
    bb, lb = 128, 512

    def body(x_ref, o_ref):
        o_ref[...] = x_ref[...].T

    return pl.pallas_call(
        body,
        grid=(ld // lb, b // bb),
        in_specs=[pl.BlockSpec((bb, lb), lambda i, j: (j, i))],
        out_specs=pl.BlockSpec((lb, bb), lambda i, j: (i, j)),
        out_shape=jax.ShapeDtypeStruct((ld, b), jnp.float32),
    )(x)


@functools.partial(jax.jit, static_argnames=("n_tok", "l_len", "d_model"))
def _sc_embed(ids_flat, table2d_in, pos2d, n_tok, l_len, d_model):
    group = l_len
    vocab = table2d_in.shape[0]
    tok_per_w = n_tok // NW
    groups_per_w = tok_per_w // group
    chunk1 = group - CHUNK0
    mesh = plsc.VectorSubcoreMesh(core_axis_name="c", subcore_axis_name="s")

    @functools.partial(
        pl.kernel,
        out_type=jax.ShapeDtypeStruct((n_tok, d_model), jnp.float32),
        mesh=mesh,
        scratch_types=(
            [pltpu.VMEM((tok_per_w,), jnp.int32),
             pltpu.VMEM((group, d_model), jnp.float32)]
            + [pltpu.VMEM((group, d_model), jnp.float32) for _ in range(NBUF)]
            + [pltpu.SemaphoreType.DMA for _ in range(2 * NBUF)]
        ),
        compiler_params=pltpu.CompilerParams(use_tc_tiling_on_sc=False,
                                             needs_layout_passes=False),
    )
    def k(ids_hbm, table_hbm, pos_hbm, out_hbm, idx_v, pos_v, *rest):
        bufs = rest[:NBUF]
        gsems = rest[NBUF:2 * NBUF]
        osems = rest[2 * NBUF:]
        wid = lax.axis_index("s") * NC + lax.axis_index("c")
        base_row = wid * tok_per_w
        table2d = table_hbm

        pltpu.sync_copy(ids_hbm.at[pl.ds(wid * tok_per_w, tok_per_w)], idx_v)
        pltpu.sync_copy(pos_hbm, pos_v)

        def start_gather(g, b):
            pltpu.async_copy(
                table2d.at[idx_v.at[pl.ds(g * group, CHUNK0)]],
                bufs[b].at[pl.ds(0, CHUNK0)],
                gsems[b])
            pltpu.async_copy(
                table2d.at[idx_v.at[pl.ds(g * group + CHUNK0, chunk1)]],
                bufs[b].at[pl.ds(CHUNK0, chunk1)],
                gsems[b])

        def wait_gather(b):
            pltpu.make_async_copy(pos_hbm, bufs[b], gsems[b]).wait()

        def wait_write(b):
            pltpu.make_async_copy(bufs[b],
                                  out_hbm.at[pl.ds(0, group)],
                                  osems[b]).wait()

        start_gather(0, 0)
        start_gather(1, 1)

        def outer(og, carry):
            for b in range(NBUF):
                g = og * NBUF + b
                wait_gather(b)

                def row_body(i, _):
                    for u in range(UNROLL):
                        r = i * UNROLL + u
                        for c in range(d_model // LANES):
                            plsc.addupdate(
                                bufs[b].at[r, pl.ds(c * LANES, LANES)],
                                pos_v[r, pl.ds(c * LANES, LANES)])
                    return 0

                lax.fori_loop(0, group // UNROLL, row_body, 0, unroll=False)
                pltpu.async_copy(
                    bufs[b],
                    out_hbm.at[pl.ds(base_row + g * group, group)],
                    osems[b])

                h = g + 2
                hb = (b + 2) % NBUF

                @pl.when(h < groups_per_w)
                def _():
                    @pl.when(h >= NBUF)
                    def _():
                        wait_write(hb)
                    start_gather(h, hb)
            return carry

        lax.fori_loop(0, groups_per_w // NBUF, outer, 0, unroll=False)

    return k(ids_flat, table2d_in, pos2d)


def kernel(input_ids, token_embed, positional):
    b, l = input_ids.shape
    vocab, d = token_embed.shape
    v = input_ids.astype(jnp.int32).reshape(-1)
    w_main = ((v & jnp.int32(-128)) + ((v & jnp.int32(63)) << 1)
              + ((v >> 6) & jnp.int32(1)))
    last_start = (vocab // 128) * 128
    if last_start < vocab:
        half = (vocab - last_start) // 2
        t = v - jnp.int32(last_start)
        w_tail = (jnp.int32(last_start) + 2 * (t % jnp.int32(half))
                  + t // jnp.int32(half))
        ids_flat = jnp.where(v < last_start, w_main, w_tail)
    else:
        ids_flat = w_main
    table_t = token_embed.T
    table_packed = _tc_pack(table_t, vocab, d)
    nfull = (vocab // VBLK) * VBLK
    if nfull < vocab:
        xt = lax.slice(table_t, (0, nfull), (d, vocab)).T
        tail = vocab - nfull
        tg = tail // 128
        parts = []
        if tg:
            g3 = xt[:tg * 128].reshape(tg, 128, d)
            parts.append(jnp.concatenate(
                [g3[:, :64, :].reshape(tg * 64, d),
                 g3[:, 64:, :].reshape(tg * 64, d)], axis=1))
        rem = tail - tg * 128
        if rem:
            xt2 = xt[tg * 128:]
            parts.append(jnp.concatenate(
                [xt2[:rem // 2], xt2[rem // 2:]], axis=1))
        tailp = jnp.concatenate(parts, axis=0)
        table_packed = lax.dynamic_update_slice(
            table_packed, tailp, (nfull // 2, 0))
    out = _sc_embed(ids_flat, table_packed.reshape(vocab, d),
                    positional[:l], b * l, l, d)
    out_t = _tc_out_t(out.reshape(b, l * d))
    return out_t.reshape(l, d, b).transpose(2, 0, 1)


# speedup vs baseline: 1.2850x; 1.2850x over previous
import functools

import jax
import jax.numpy as jnp
from jax import lax
from jax.experimental import pallas as pl
from jax.experimental.pallas import tpu as pltpu
from jax.experimental.pallas import tpu_sc as plsc

LANES = 16
NC, NS = 2, 16
NW = NC * NS
CHUNK0 = 128
NBUF = 4
UNROLL = 8
VBLK = 8192


@functools.partial(jax.jit, static_argnames=("vocab", "d_model"))
def _tc_pack(table_t, vocab, d_model):
    per = 128 // d_model
    grid = vocab // VBLK

    def body(x_ref, o_ref):
        xt = x_ref[...].T
        v3 = xt.reshape(VBLK // 128, 128, d_model)
        a = v3[:, :64, :].reshape(VBLK // per, d_model)
        b = v3[:, 64:, :].reshape(VBLK // per, d_model)
        o_ref[...] = jnp.concatenate([a, b], axis=1)

    return pl.pallas_call(
        body,
        grid=(grid,),
        in_specs=[pl.BlockSpec((d_model, VBLK), lambda i: (0, i))],
        out_specs=pl.BlockSpec((VBLK // per, 128), lambda i: (i, 0)),
        out_shape=jax.ShapeDtypeStruct((vocab // per, 128), jnp.float32),
    )(table_t)


@functools.partial(jax.jit, static_argnames=("n_tok", "l_len", "d_model"))
def _sc_embed(ids_flat, table2d_in, pos2d, n_tok, l_len, d_model):
    group = l_len
    vocab = table2d_in.shape[0]
    tok_per_w = n_tok // NW
    groups_per_w = tok_per_w // group
    chunk1 = group - CHUNK0
    mesh = plsc.VectorSubcoreMesh(core_axis_name="c", subcore_axis_name="s")

    @functools.partial(
        pl.kernel,
        out_type=jax.ShapeDtypeStruct((n_tok // l_len, l_len, d_model),
                                      jnp.float32),
        mesh=mesh,
        scratch_types=(
            [pltpu.VMEM((tok_per_w,), jnp.int32),
             pltpu.VMEM((group, d_model), jnp.float32)]
            + [pltpu.VMEM((group, d_model), jnp.float32) for _ in range(NBUF)]
            + [pltpu.SemaphoreType.DMA for _ in range(2 * NBUF)]
        ),
        compiler_params=pltpu.CompilerParams(use_tc_tiling_on_sc=False,
                                             needs_layout_passes=False),
    )
    def k(ids_hbm, table_hbm, pos_hbm, out_hbm, idx_v, pos_v, *rest):
        bufs = rest[:NBUF]
        gsems = rest[NBUF:2 * NBUF]
        osems = rest[2 * NBUF:]
        wid = lax.axis_index("s") * NC + lax.axis_index("c")
        base_row = wid * tok_per_w
        table2d = table_hbm

        pltpu.sync_copy(ids_hbm.at[pl.ds(wid * tok_per_w, tok_per_w)], idx_v)
        pltpu.sync_copy(pos_hbm, pos_v)

        def start_gather(g, b):
            pltpu.async_copy(
                table2d.at[idx_v.at[pl.ds(g * group, CHUNK0)]],
                bufs[b].at[pl.ds(0, CHUNK0)],
                gsems[b])
            pltpu.async_copy(
                table2d.at[idx_v.at[pl.ds(g * group + CHUNK0, chunk1)]],
                bufs[b].at[pl.ds(CHUNK0, chunk1)],
                gsems[b])

        def wait_gather(b):
            pltpu.make_async_copy(pos_hbm, bufs[b], gsems[b]).wait()

        def wait_write(b):
            pltpu.make_async_copy(bufs[b], out_hbm.at[0], osems[b]).wait()

        start_gather(0, 0)
        start_gather(1, 1)

        def outer(og, carry):
            for b in range(NBUF):
                g = og * NBUF + b
                wait_gather(b)

                def row_body(i, _):
                    for u in range(UNROLL):
                        r = i * UNROLL + u
                        for c in range(d_model // LANES):
                            plsc.addupdate(
                                bufs[b].at[r, pl.ds(c * LANES, LANES)],
                                pos_v[r, pl.ds(c * LANES, LANES)])
                    return 0

                lax.fori_loop(0, group // UNROLL, row_body, 0, unroll=False)
                pltpu.async_copy(bufs[b],
                                 out_hbm.at[wid * groups_per_w + g],
                                 osems[b])

                h = g + 2
                hb = (b + 2) % NBUF

                @pl.when(h < groups_per_w)
                def _():
                    @pl.when(h >= NBUF)
                    def _():
                        wait_write(hb)
                    start_gather(h, hb)
            return carry

        lax.fori_loop(0, groups_per_w // NBUF, outer, 0, unroll=False)

    return k(ids_flat, table2d_in, pos2d)


def kernel(input_ids, token_embed, positional):
    b, l = input_ids.shape
    vocab, d = token_embed.shape
    v = input_ids.astype(jnp.int32).reshape(-1)
    w_main = ((v & jnp.int32(-128)) + ((v & jnp.int32(63)) << 1)
              + ((v >> 6) & jnp.int32(1)))
    last_start = (vocab // 128) * 128
    if last_start < vocab:
        half = (vocab - last_start) // 2
        t = v - jnp.int32(last_start)
        w_tail = (jnp.int32(last_start) + 2 * (t % jnp.int32(half))
                  + t // jnp.int32(half))
        ids_flat = jnp.where(v < last_start, w_main, w_tail)
    else:
        ids_flat = w_main
    table_t = token_embed.T
    table_packed = _tc_pack(table_t, vocab, d)
    nfull = (vocab // VBLK) * VBLK
    if nfull < vocab:
        xt = lax.slice(table_t, (0, nfull), (d, vocab)).T
        tail = vocab - nfull
        tg = tail // 128
        parts = []
        if tg:
            g3 = xt[:tg * 128].reshape(tg, 128, d)
            parts.append(jnp.concatenate(
                [g3[:, :64, :].reshape(tg * 64, d),
                 g3[:, 64:, :].reshape(tg * 64, d)], axis=1))
        rem = tail - tg * 128
        if rem:
            xt2 = xt[tg * 128:]
            parts.append(jnp.concatenate(
                [xt2[:rem // 2], xt2[rem // 2:]], axis=1))
        tailp = jnp.concatenate(parts, axis=0)
        table_packed = lax.dynamic_update_slice(
            table_packed, tailp, (nfull // 2, 0))
    return _sc_embed(ids_flat, table_packed.reshape(vocab, d),
                     positional[:l], b * l, l, d)


# speedup vs baseline: 1.3917x; 1.0830x over previous
import functools

import jax
import jax.numpy as jnp
from jax import lax
from jax.experimental import pallas as pl
from jax.experimental.pallas import tpu as pltpu
from jax.experimental.pallas import tpu_sc as plsc

LANES = 16
NC, NS = 2, 16
NW = NC * NS
CHUNK0 = 128
NBUF = 4
UNROLL = 8
VBLK = 16384


@functools.partial(jax.jit, static_argnames=("vocab", "d_model"))
def _tc_pack(table_t, vocab, d_model):
    per = 128 // d_model
    grid = vocab // VBLK

    def body(x_ref, o_ref):
        xt = x_ref[...].T
        v3 = xt.reshape(VBLK // 128, 128, d_model)
        a = v3[:, :64, :].reshape(VBLK // per, d_model)
        b = v3[:, 64:, :].reshape(VBLK // per, d_model)
        o_ref[...] = jnp.concatenate([a, b], axis=1)

    return pl.pallas_call(
        body,
        grid=(grid,),
        in_specs=[pl.BlockSpec((d_model, VBLK), lambda i: (0, i))],
        out_specs=pl.BlockSpec((VBLK // per, 128), lambda i: (i, 0)),
        out_shape=jax.ShapeDtypeStruct((vocab // per, 128), jnp.float32),
    )(table_t)


@functools.partial(jax.jit, static_argnames=("n_tok", "l_len", "d_model"))
def _sc_embed(ids_flat, table2d_in, pos2d, n_tok, l_len, d_model):
    group = l_len
    vocab = table2d_in.shape[0]
    tok_per_w = n_tok // NW
    groups_per_w = tok_per_w // group
    chunk1 = group - CHUNK0
    mesh = plsc.VectorSubcoreMesh(core_axis_name="c", subcore_axis_name="s")

    @functools.partial(
        pl.kernel,
        out_type=jax.ShapeDtypeStruct((n_tok // l_len, l_len, d_model),
                                      jnp.float32),
        mesh=mesh,
        scratch_types=(
            [pltpu.VMEM((tok_per_w,), jnp.int32),
             pltpu.VMEM((group, d_model), jnp.float32)]
            + [pltpu.VMEM((group, d_model), jnp.float32) for _ in range(NBUF)]
            + [pltpu.SemaphoreType.DMA for _ in range(2 * NBUF)]
        ),
        compiler_params=pltpu.CompilerParams(use_tc_tiling_on_sc=False,
                                             needs_layout_passes=False),
    )
    def k(ids_hbm, table_hbm, pos_hbm, out_hbm, idx_v, pos_v, *rest):
        bufs = rest[:NBUF]
        gsems = rest[NBUF:2 * NBUF]
        osems = rest[2 * NBUF:]
        wid = lax.axis_index("s") * NC + lax.axis_index("c")
        base_row = wid * tok_per_w
        table2d = table_hbm

        pltpu.sync_copy(ids_hbm.at[pl.ds(wid * tok_per_w, tok_per_w)], idx_v)
        pltpu.sync_copy(pos_hbm, pos_v)

        def start_gather(g, b):
            pltpu.async_copy(
                table2d.at[idx_v.at[pl.ds(g * group, CHUNK0)]],
                bufs[b].at[pl.ds(0, CHUNK0)],
                gsems[b])
            pltpu.async_copy(
                table2d.at[idx_v.at[pl.ds(g * group + CHUNK0, chunk1)]],
                bufs[b].at[pl.ds(CHUNK0, chunk1)],
                gsems[b])

        def wait_gather(b):
            pltpu.make_async_copy(pos_hbm, bufs[b], gsems[b]).wait()

        def wait_write(b):
            pltpu.make_async_copy(bufs[b], out_hbm.at[0], osems[b]).wait()

        start_gather(0, 0)
        start_gather(1, 1)

        def outer(og, carry):
            for b in range(NBUF):
                g = og * NBUF + b
                wait_gather(b)

                def row_body(i, _):
                    for u in range(UNROLL):
                        r = i * UNROLL + u
                        for c in range(d_model // LANES):
                            plsc.addupdate(
                                bufs[b].at[r, pl.ds(c * LANES, LANES)],
                                pos_v[r, pl.ds(c * LANES, LANES)])
                    return 0

                lax.fori_loop(0, group // UNROLL, row_body, 0, unroll=False)
                pltpu.async_copy(bufs[b],
                                 out_hbm.at[wid * groups_per_w + g],
                                 osems[b])

                h = g + 2
                hb = (b + 2) % NBUF

                @pl.when(h < groups_per_w)
                def _():
                    @pl.when(h >= NBUF)
                    def _():
                        wait_write(hb)
                    start_gather(h, hb)
            return carry

        lax.fori_loop(0, groups_per_w // NBUF, outer, 0, unroll=False)

    return k(ids_flat, table2d_in, pos2d)


def kernel(input_ids, token_embed, positional):
    b, l = input_ids.shape
    vocab, d = token_embed.shape
    v = input_ids.astype(jnp.int32).reshape(-1)
    w_main = ((v & jnp.int32(-128)) + ((v & jnp.int32(63)) << 1)
              + ((v >> 6) & jnp.int32(1)))
    last_start = (vocab // 128) * 128
    if last_start < vocab:
        half = (vocab - last_start) // 2
        t = v - jnp.int32(last_start)
        w_tail = (jnp.int32(last_start) + 2 * (t % jnp.int32(half))
                  + t // jnp.int32(half))
        ids_flat = jnp.where(v < last_start, w_main, w_tail)
    else:
        ids_flat = w_main
    table_t = token_embed.T
    table_packed = _tc_pack(table_t, vocab, d)
    nfull = (vocab // VBLK) * VBLK
    if nfull < vocab:
        xt = lax.slice(table_t, (0, nfull), (d, vocab)).T
        tail = vocab - nfull
        tg = tail // 128
        parts = []
        if tg:
            g3 = xt[:tg * 128].reshape(tg, 128, d)
            parts.append(jnp.concatenate(
                [g3[:, :64, :].reshape(tg * 64, d),
                 g3[:, 64:, :].reshape(tg * 64, d)], axis=1))
        rem = tail - tg * 128
        if rem:
            xt2 = xt[tg * 128:]
            parts.append(jnp.concatenate(
                [xt2[:rem // 2], xt2[rem // 2:]], axis=1))
        tailp = jnp.concatenate(parts, axis=0)
        table_packed = lax.dynamic_update_slice(
            table_packed, tailp, (nfull // 2, 0))
    return _sc_embed(ids_flat, table_packed.reshape(vocab, d),
                     positional[:l], b * l, l, d)
